# baseline (speedup 1.0000x reference)
import jax
import jax.numpy as jnp
from jax import lax
from jax.experimental import pallas as pl
from jax.experimental.pallas import tpu as pltpu
from jax.experimental.pallas import tpu_sc as plsc

_K = 64
_B = 128
_N = 32768
_NV = _N // 16
_NC = 2
_NS = 16
_ROWS_PER_W = _B // (_NC * _NS)
_CAP = 16384


def _suffix(v):
    return lax.rev(plsc.cumsum(lax.rev(v, (0,))), (0,))


def _extract(vec, i):
    lane = jnp.arange(16, dtype=jnp.int32)
    return jnp.sum(jnp.where(lane == i, vec, 0))


def _level_select(hist_ref, k_rem):
    tot = jnp.zeros((16,), jnp.int32)
    for r in range(16):
        tot = tot + hist_ref[pl.ds(r * 16, 16)]
    s = _suffix(tot)
    c0 = jnp.sum((s >= k_rem).astype(jnp.int32)) - 1
    above_chunks = _extract(s, c0) - _extract(tot, c0)
    k2 = k_rem - above_chunks
    lane = jnp.arange(16, dtype=jnp.int32)
    bvec = plsc.load_gather(hist_ref, [lane * 16 + c0])
    sb = _suffix(bvec)
    r0 = jnp.sum((sb >= k2).astype(jnp.int32)) - 1
    sb_r0 = _extract(sb, r0)
    bv_r0 = _extract(bvec, r0)
    d0 = c0 * 16 + r0
    k_next = k2 - (sb_r0 - bv_r0)
    return d0, k_next, bv_r0


def _clear_hist(hist):
    z = jnp.zeros((16,), jnp.int32)
    for r in range(16):
        hist[pl.ds(r * 16, 16)] = z


def _hist_byte(hist, k, shift_hi, shift_d, prefix, extra_mask=None):
    m = lax.shift_right_arithmetic(k, shift_hi) == prefix
    if extra_mask is not None:
        m = m & extra_mask
    d = lax.shift_right_logical(k, shift_d) & 0xFF
    idx = (d & 15) * 16 + lax.shift_right_logical(d, 4)
    cnt, last = plsc.scan_count(idx, mask=m)
    plsc.addupdate_scatter(hist, [idx], cnt, mask=last)


def _sc_body(x_hbm, o_hbm, buf0, buf1, keyb, candb, hist, si0, si1, so0, so1):
    wid = lax.axis_index("s") * _NC + lax.axis_index("c")
    lane = jnp.arange(16, dtype=jnp.int32)
    base = wid * _ROWS_PER_W
    bufs = (buf0, buf1)
    sin = (si0, si1)
    sout = (so0, so1)

    def cp_in(j, b):
        return pltpu.make_async_copy(x_hbm.at[base + j], bufs[b], sin[b])

    def cp_out(j, b):
        return pltpu.make_async_copy(bufs[b], o_hbm.at[base + j], sout[b])

    def process_row(buf):
        _clear_hist(hist)

        @plsc.parallel_loop(0, _N, step=16, unroll=8)
        def _p1(i):
            xv = buf[pl.ds(i, 16)]
            v = lax.bitcast_convert_type(xv, jnp.int32)
            k = jnp.where(v >= 0, v, v ^ 0x7FFFFFFF)
            keyb[pl.ds(i, 16)] = k
            d = lax.shift_right_arithmetic(k, 24) + 128
            idx = (d & 15) * 16 + lax.shift_right_logical(d, 4)
            cnt, last = plsc.scan_count(idx)
            plsc.addupdate_scatter(hist, [idx], cnt, mask=last)

        d1, k_rem1, n1 = _level_select(hist, jnp.int32(_K))
        prefix1 = d1 - 128

        def compact_path(_):
            zoff = jnp.zeros((16,), jnp.int32)

            @plsc.parallel_loop(0, _N, step=16, unroll=8, carry=zoff)
            def _p2(i, off):
                k = keyb[pl.ds(i, 16)]
                m = lax.shift_right_arithmetic(k, 24) == prefix1
                mi = m.astype(jnp.int32)
                pos = off + plsc.cumsum(mi) - 1
                plsc.store_scatter(candb, [pos], k, mask=m)
                return off + plsc.all_reduce_population_count(m)

            n1r = lax.shift_left(lax.shift_right_logical(n1 + 15, 4), 4)
            k_rem = k_rem1
            prefix = prefix1
            n_eq = n1
            for shift_hi, shift_d in ((24, 16), (16, 8), (8, 0)):
                _clear_hist(hist)

                @plsc.parallel_loop(0, n1r, step=16)
                def _ml(i, shift_hi=shift_hi, shift_d=shift_d, prefix=prefix):
                    k = candb[pl.ds(i, 16)]
                    valid = (i + lane) < n1
                    _hist_byte(hist, k, shift_hi, shift_d, prefix, valid)

                dl, k_rem, n_eq = _level_select(hist, k_rem)
                prefix = prefix * 256 + dl
            return prefix, k_rem, n_eq

        def full_path(_):
            k_rem = k_rem1
            prefix = prefix1
            n_eq = n1
            for shift_hi, shift_d in ((24, 16), (16, 8), (8, 0)):
                _clear_hist(hist)

                @plsc.parallel_loop(0, _N, step=16, unroll=8)
                def _pm(i, shift_hi=shift_hi, shift_d=shift_d, prefix=prefix):
                    k = keyb[pl.ds(i, 16)]
                    _hist_byte(hist, k, shift_hi, shift_d, prefix)

                dl, k_rem, n_eq = _level_select(hist, k_rem)
                prefix = prefix * 256 + dl
            return prefix, k_rem, n_eq

        t, m_keep, n_eq = lax.cond(n1 <= _CAP, compact_path, full_path, 0)

        @plsc.parallel_loop(0, _N, step=16, unroll=8)
        def _pfin(i):
            k = keyb[pl.ds(i, 16)]
            v = jnp.where(k >= 0, k, k ^ 0x7FFFFFFF)
            xv = lax.bitcast_convert_type(v, jnp.float32)
            buf[pl.ds(i, 16)] = jnp.where(k >= t, xv, 0.0)

        @pl.when(n_eq > m_keep)
        def _fix():
            def fb(i, cnt):
                k = keyb[pl.ds(i * 16, 16)]
                eq = k == t
                eqi = eq.astype(jnp.int32)
                rank = cnt + plsc.cumsum(eqi) - 1
                kill = eq & (rank >= m_keep)
                xv = buf[pl.ds(i * 16, 16)]
                buf[pl.ds(i * 16, 16)] = jnp.where(kill, 0.0, xv)
                return cnt + jnp.sum(eqi)

            lax.fori_loop(0, _NV, fb, jnp.int32(0))

    cp_in(0, 0).start()
    cp_in(1, 1).start()
    for j in range(_ROWS_PER_W):
        b = j % 2
        cp_in(j, b).wait()
        process_row(bufs[b])
        cp_out(j, b).start()
        if j + 2 < _ROWS_PER_W:
            cp_out(j, b).wait()
            cp_in(j + 2, b).start()
    cp_out(_ROWS_PER_W - 2, _ROWS_PER_W % 2).wait()
    cp_out(_ROWS_PER_W - 1, 1 - _ROWS_PER_W % 2).wait()


def _make(interpret=False):
    mesh = plsc.VectorSubcoreMesh(core_axis_name="c", subcore_axis_name="s")
    return pl.kernel(
        _sc_body,
        out_type=jax.ShapeDtypeStruct((_B, _N), jnp.float32),
        mesh=mesh,
        scratch_types=[
            pltpu.VMEM((_N,), jnp.float32),
            pltpu.VMEM((_N,), jnp.float32),
            pltpu.VMEM((_N,), jnp.int32),
            pltpu.VMEM((_CAP,), jnp.int32),
            pltpu.VMEM((256,), jnp.int32),
            pltpu.SemaphoreType.DMA,
            pltpu.SemaphoreType.DMA,
            pltpu.SemaphoreType.DMA,
            pltpu.SemaphoreType.DMA,
        ],
        compiler_params=pltpu.CompilerParams(needs_layout_passes=False),
        interpret=interpret,
    )


def kernel(x):
    return _make()(x)


# speedup vs baseline: 1.1074x; 1.1074x over previous
import jax
import jax.numpy as jnp
from jax import lax
from jax.experimental import pallas as pl
from jax.experimental.pallas import tpu as pltpu
from jax.experimental.pallas import tpu_sc as plsc

_K = 64
_B = 128
_N = 32768
_NV = _N // 16
_NC = 2
_NS = 16
_ROWS_PER_W = _B // (_NC * _NS)
_CAP = 16384


def _suffix(v):
    return lax.rev(plsc.cumsum(lax.rev(v, (0,))), (0,))


def _extract(vec, i):
    lane = jnp.arange(16, dtype=jnp.int32)
    return jnp.sum(jnp.where(lane == i, vec, 0))


def _level_select(hist_ref, k_rem):
    lane = jnp.arange(16, dtype=jnp.int32)
    tot = jnp.zeros((16,), jnp.int32)
    for r in range(16):
        tot = tot + plsc.load_gather(hist_ref, [lane * 16 + r])
    s = _suffix(tot)
    c0 = jnp.sum((s >= k_rem).astype(jnp.int32)) - 1
    above_chunks = _extract(s, c0) - _extract(tot, c0)
    k2 = k_rem - above_chunks
    bvec = plsc.load_gather(hist_ref, [c0 * 16 + lane])
    sb = _suffix(bvec)
    r0 = jnp.sum((sb >= k2).astype(jnp.int32)) - 1
    sb_r0 = _extract(sb, r0)
    bv_r0 = _extract(bvec, r0)
    d0 = c0 * 16 + r0
    k_next = k2 - (sb_r0 - bv_r0)
    return d0, k_next, bv_r0


def _clear_hist(hist):
    z = jnp.zeros((16,), jnp.int32)
    for r in range(16):
        hist[pl.ds(r * 16, 16)] = z


def _hist_byte(hist, k, shift_hi, shift_d, prefix, extra_mask=None):
    m = lax.shift_right_arithmetic(k, shift_hi) == prefix
    if extra_mask is not None:
        m = m & extra_mask
    idx = lax.shift_right_logical(k, shift_d) & 0xFF
    cnt, last = plsc.scan_count(idx, mask=m)
    plsc.addupdate_scatter(hist, [idx], cnt, mask=last)


def _sc_body(x_hbm, o_hbm, buf, keyb, candb, hist):
    wid = lax.axis_index("s") * _NC + lax.axis_index("c")
    lane = jnp.arange(16, dtype=jnp.int32)

    def do_row(j, carry):
        row = wid * _ROWS_PER_W + j
        pltpu.sync_copy(x_hbm.at[row], buf)
        _clear_hist(hist)

        @plsc.parallel_loop(0, _N, step=16, unroll=8)
        def _p1(i):
            xv = buf[pl.ds(i, 16)]
            v = lax.bitcast_convert_type(xv, jnp.int32)
            k = jnp.where(v >= 0, v, v ^ 0x7FFFFFFF)
            keyb[pl.ds(i, 16)] = k
            idx = lax.shift_right_arithmetic(k, 24) + 128
            cnt, last = plsc.scan_count(idx)
            plsc.addupdate_scatter(hist, [idx], cnt, mask=last)

        d1, k_rem1, n1 = _level_select(hist, jnp.int32(_K))
        prefix1 = d1 - 128

        def compact_path(_):
            zoff = jnp.full((16,), -1, jnp.int32)

            @plsc.parallel_loop(0, _N, step=16, unroll=8, carry=zoff)
            def _p2(i, off):
                k = keyb[pl.ds(i, 16)]
                m = lax.shift_right_arithmetic(k, 24) == prefix1
                mi = m.astype(jnp.int32)
                pos = off + plsc.cumsum(mi)
                plsc.store_scatter(candb, [pos], k, mask=m)
                return off + plsc.all_reduce_population_count(m)

            n1r = lax.shift_left(lax.shift_right_logical(n1 + 15, 4), 4)
            k_rem = k_rem1
            prefix = prefix1
            n_eq = n1
            for shift_hi, shift_d in ((24, 16), (16, 8), (8, 0)):
                _clear_hist(hist)

                @plsc.parallel_loop(0, n1r, step=16)
                def _ml(i, shift_hi=shift_hi, shift_d=shift_d, prefix=prefix):
                    k = candb[pl.ds(i, 16)]
                    valid = (i + lane) < n1
                    _hist_byte(hist, k, shift_hi, shift_d, prefix, valid)

                dl, k_rem, n_eq = _level_select(hist, k_rem)
                prefix = prefix * 256 + dl
            return prefix, k_rem, n_eq

        def full_path(_):
            k_rem = k_rem1
            prefix = prefix1
            n_eq = n1
            for shift_hi, shift_d in ((24, 16), (16, 8), (8, 0)):
                _clear_hist(hist)

                @plsc.parallel_loop(0, _N, step=16, unroll=8)
                def _pm(i, shift_hi=shift_hi, shift_d=shift_d, prefix=prefix):
                    k = keyb[pl.ds(i, 16)]
                    _hist_byte(hist, k, shift_hi, shift_d, prefix)

                dl, k_rem, n_eq = _level_select(hist, k_rem)
                prefix = prefix * 256 + dl
            return prefix, k_rem, n_eq

        t, m_keep, n_eq = lax.cond(n1 <= _CAP, compact_path, full_path, 0)

        @plsc.parallel_loop(0, _N, step=16, unroll=8)
        def _pfin(i):
            k = keyb[pl.ds(i, 16)]
            v = jnp.where(k >= 0, k, k ^ 0x7FFFFFFF)
            xv = lax.bitcast_convert_type(v, jnp.float32)
            buf[pl.ds(i, 16)] = jnp.where(k >= t, xv, 0.0)

        @pl.when(n_eq > m_keep)
        def _fix():
            def fb(i, cnt):
                k = keyb[pl.ds(i * 16, 16)]
                eq = k == t
                eqi = eq.astype(jnp.int32)
                rank = cnt + plsc.cumsum(eqi) - 1
                kill = eq & (rank >= m_keep)
                xv = buf[pl.ds(i * 16, 16)]
                buf[pl.ds(i * 16, 16)] = jnp.where(kill, 0.0, xv)
                return cnt + jnp.sum(eqi)

            lax.fori_loop(0, _NV, fb, jnp.int32(0))

        pltpu.sync_copy(buf, o_hbm.at[row])
        return carry

    lax.fori_loop(0, _ROWS_PER_W, do_row, 0)


def _make(interpret=False):
    mesh = plsc.VectorSubcoreMesh(core_axis_name="c", subcore_axis_name="s")
    return pl.kernel(
        _sc_body,
        out_type=jax.ShapeDtypeStruct((_B, _N), jnp.float32),
        mesh=mesh,
        scratch_types=[
            pltpu.VMEM((_N,), jnp.float32),
            pltpu.VMEM((_N,), jnp.int32),
            pltpu.VMEM((_CAP,), jnp.int32),
            pltpu.VMEM((256,), jnp.int32),
        ],
        compiler_params=pltpu.CompilerParams(needs_layout_passes=False),
        interpret=interpret,
    )


def kernel(x):
    return _make()(x)
